# baseline (speedup 1.0000x reference)
import functools

import jax
import jax.numpy as jnp
from jax import lax
from jax.experimental import pallas as pl
from jax.experimental.pallas import tpu as pltpu
from jax.experimental.pallas import tpu_sc as plsc

N = 10000
E = 320000
D = 128
G = 128

NC = 2
NS = 16
NW = NC * NS
CH = 64
NCH = 164
EPT = NCH * CH
EPAD = NW * EPT
NSL = 4
NPAIR = NCH // NSL
NPAD = 10240
ZPT = NPAD // NS
ZCH = ZPT // CH

_sc_mesh = plsc.VectorSubcoreMesh(
    core_axis_name="c", subcore_axis_name="s", num_cores=NC, num_subcores=NS)


@functools.partial(
    pl.kernel,
    out_type=jax.ShapeDtypeStruct((NC, NPAD, D), jnp.float32),
    mesh=_sc_mesh,
    scratch_types=[
        pltpu.VMEM((NSL, CH), jnp.int32),
        pltpu.VMEM((NSL, CH), jnp.int32),
        pltpu.VMEM((NSL, CH, D), jnp.float32),
        pltpu.VMEM_SHARED((NPAD, D), jnp.float32),
    ] + [pltpu.SemaphoreType.DMA] * (4 * NSL),
)
def _edge_agg(h_hbm, src_hbm, dst_hbm, out_hbm, sring, dring, rows_v,
              acc_sh, *sems):
    is_sem = sems[:NSL]
    id_sem = sems[NSL:2 * NSL]
    gsem = sems[2 * NSL:3 * NSL]
    ssem = sems[3 * NSL:]
    cid = lax.axis_index("c")
    sid = lax.axis_index("s")
    wid = sid * NC + cid
    base = wid * EPT

    def issue_src(c, j):
        pltpu.async_copy(src_hbm.at[pl.ds(base + c * CH, CH)], sring.at[j],
                         is_sem[j])

    def issue_dst(c, j):
        pltpu.async_copy(dst_hbm.at[pl.ds(base + c * CH, CH)], dring.at[j],
                         id_sem[j])

    def wait_src(j):
        pltpu.make_async_copy(src_hbm.at[pl.ds(0, CH)], sring.at[j],
                              is_sem[j]).wait()

    def wait_dst(j):
        pltpu.make_async_copy(dst_hbm.at[pl.ds(0, CH)], dring.at[j],
                              id_sem[j]).wait()

    def issue_gather(j):
        pltpu.async_copy(h_hbm.at[sring.at[j]], rows_v.at[j], gsem[j])

    def wait_gather(j):
        pltpu.make_async_copy(h_hbm.at[sring.at[0]], rows_v.at[j],
                              gsem[j]).wait()

    def issue_scatter(j):
        pltpu.async_copy(rows_v.at[j], acc_sh.at[dring.at[j]], ssem[j],
                         add=True)

    def wait_scatter(j):
        pltpu.make_async_copy(rows_v.at[j], acc_sh.at[dring.at[j]],
                              ssem[j]).wait()

    for j in range(NSL):
        issue_src(j, j)
    for j in (0, 1):
        issue_dst(j, j)

    zeros16 = jnp.zeros((16,), jnp.float32)

    @pl.loop(0, CH)
    def _zero_rows(rr):
        @pl.loop(0, D // 16)
        def _zero_cols(cc):
            rows_v[0, rr, pl.ds(cc * 16, 16)] = zeros16

    @pl.loop(0, ZCH)
    def _zero_acc(z):
        pltpu.sync_copy(rows_v.at[0], acc_sh.at[pl.ds(sid * ZPT + z * CH, CH)])

    plsc.subcore_barrier()

    for j in (0, 1):
        wait_src(j)
        issue_gather(j)

    for j in (0, 1):
        wait_gather(j)
        issue_src(4 + j, j)
        wait_dst(j)
        issue_scatter(j)
    for j in (2, 3):
        issue_dst(j, j)
        wait_src(j)
        issue_gather(j)
    for j in (2, 3):
        wait_gather(j)
        issue_src(4 + j, j)
        wait_dst(j)
        issue_scatter(j)
    for j in (0, 1):
        wait_scatter(j)
        issue_dst(4 + j, j)
        wait_src(j)
        issue_gather(j)

    @pl.loop(1, NPAIR - 1)
    def _pairs(p):
        c0 = p * NSL
        for j in (0, 1):
            wait_gather(j)
            issue_src(c0 + 4 + j, j)
            wait_dst(j)
            issue_scatter(j)
        for j in (2, 3):
            wait_scatter(j)
            issue_dst(c0 + j, j)
            wait_src(j)
            issue_gather(j)
        for j in (2, 3):
            wait_gather(j)
            issue_src(c0 + 4 + j, j)
            wait_dst(j)
            issue_scatter(j)
        for j in (0, 1):
            wait_scatter(j)
            issue_dst(c0 + 4 + j, j)
            wait_src(j)
            issue_gather(j)

    c0 = NCH - NSL
    for j in (0, 1):
        wait_gather(j)
        wait_dst(j)
        issue_scatter(j)
    for j in (2, 3):
        wait_scatter(j)
        issue_dst(c0 + j, j)
        wait_src(j)
        issue_gather(j)
    for j in (2, 3):
        wait_gather(j)
        wait_dst(j)
        issue_scatter(j)
    for j in range(NSL):
        wait_scatter(j)

    plsc.subcore_barrier()

    pltpu.sync_copy(acc_sh.at[pl.ds(sid * ZPT, ZPT)],
                    out_hbm.at[cid, pl.ds(sid * ZPT, ZPT)])


BN = 2000
NB = N // BN


def _layer_body(h_ref, p0_ref, p1_ref, w_ref, b_ref, o_ref):
    s = h_ref[...] + p0_ref[...] + p1_ref[...]
    y = jnp.dot(s, w_ref[...], preferred_element_type=jnp.float32) + b_ref[...]
    o_ref[...] = jnp.maximum(y, 0.0)


def _layer_tc(h, p0, p1, W, b2d):
    return pl.pallas_call(
        _layer_body,
        grid=(NB,),
        in_specs=[
            pl.BlockSpec((BN, D), lambda i: (i, 0)),
            pl.BlockSpec((BN, D), lambda i: (i, 0)),
            pl.BlockSpec((BN, D), lambda i: (i, 0)),
            pl.BlockSpec((D, D), lambda i: (0, 0)),
            pl.BlockSpec((1, D), lambda i: (0, 0)),
        ],
        out_specs=pl.BlockSpec((BN, D), lambda i: (i, 0)),
        out_shape=jax.ShapeDtypeStruct((N, D), jnp.float32),
    )(h, p0, p1, W, b2d)


def _final_body(h_ref, p0_ref, p1_ref, w2_ref, b2_ref, batch_ref, wg_ref,
                bg_ref, o_ref, sums, counts):
    i = pl.program_id(0)

    @pl.when(i == 0)
    def _():
        sums[...] = jnp.zeros_like(sums)
        counts[...] = jnp.zeros_like(counts)

    s = h_ref[...] + p0_ref[...] + p1_ref[...]
    h2 = jnp.maximum(
        jnp.dot(s, w2_ref[...], preferred_element_type=jnp.float32)
        + b2_ref[...], 0.0)

    bt = batch_ref[...].reshape(1, BN)
    gidx = lax.broadcasted_iota(jnp.int32, (G, BN), 0)
    P = (bt == gidx).astype(jnp.float32)
    sums[...] += jnp.dot(P, h2, preferred_element_type=jnp.float32)
    counts[...] += jnp.broadcast_to(jnp.sum(P, axis=1, keepdims=True), (G, D))

    @pl.when(i == NB - 1)
    def _():
        hg = sums[...] / jnp.maximum(counts[...], 1.0)
        o_ref[...] = (jnp.dot(hg, wg_ref[...], preferred_element_type=jnp.float32)
                      + bg_ref[...])


def _final_tc(h1, p0, p1, W2, b2d, batch3d, Wg, bg2d):
    return pl.pallas_call(
        _final_body,
        grid=(NB,),
        in_specs=[
            pl.BlockSpec((BN, D), lambda i: (i, 0)),
            pl.BlockSpec((BN, D), lambda i: (i, 0)),
            pl.BlockSpec((BN, D), lambda i: (i, 0)),
            pl.BlockSpec((D, D), lambda i: (0, 0)),
            pl.BlockSpec((1, D), lambda i: (0, 0)),
            pl.BlockSpec((1, 1, BN), lambda i: (i, 0, 0)),
            pl.BlockSpec((D, D), lambda i: (0, 0)),
            pl.BlockSpec((1, D), lambda i: (0, 0)),
        ],
        out_specs=pl.BlockSpec((G, D), lambda i: (0, 0)),
        out_shape=jax.ShapeDtypeStruct((G, D), jnp.float32),
        scratch_shapes=[
            pltpu.VMEM((G, D), jnp.float32),
            pltpu.VMEM((G, D), jnp.float32),
        ],
    )(h1, p0, p1, W2, b2d, batch3d, Wg, bg2d)


def kernel(x, edge_index, batch, W1, b1, W2, b2, Wg, bg):
    pad = EPAD - E
    src = jnp.concatenate(
        [edge_index[0].astype(jnp.int32), jnp.zeros((pad,), jnp.int32)])
    dst = jnp.concatenate(
        [edge_index[1].astype(jnp.int32),
         jnp.full((pad,), NPAD - 1, jnp.int32)])
    batch3d = batch.astype(jnp.int32).reshape(NB, 1, BN)

    p = _edge_agg(x, src, dst)
    h1 = _layer_tc(x, p[0, :N], p[1, :N], W1, b1.reshape(1, D))
    q = _edge_agg(h1, src, dst)
    return _final_tc(h1, q[0, :N], q[1, :N], W2, b2.reshape(1, D),
                     batch3d, Wg, bg.reshape(1, D))


# speedup vs baseline: 1.0001x; 1.0001x over previous
import functools

import jax
import jax.numpy as jnp
from jax import lax
from jax.experimental import pallas as pl
from jax.experimental.pallas import tpu as pltpu
from jax.experimental.pallas import tpu_sc as plsc

N = 10000
E = 320000
D = 128
G = 128

NC = 2
NS = 16
NW = NC * NS
CH = 64
NCH = 164
EPT = NCH * CH
EPAD = NW * EPT
NSL = 4
NPAIR = NCH // NSL
NPAD = 10240
ZPT = NPAD // NS
ZCH = ZPT // CH

_sc_mesh = plsc.VectorSubcoreMesh(
    core_axis_name="c", subcore_axis_name="s", num_cores=NC, num_subcores=NS)


@functools.partial(
    pl.kernel,
    out_type=jax.ShapeDtypeStruct((NC, NPAD, D), jnp.float32),
    mesh=_sc_mesh,
    scratch_types=[
        pltpu.VMEM((NSL, CH), jnp.int32),
        pltpu.VMEM((NSL, CH), jnp.int32),
        pltpu.VMEM((NSL, CH, D), jnp.float32),
        pltpu.VMEM_SHARED((NPAD, D), jnp.float32),
    ] + [pltpu.SemaphoreType.DMA] * (4 * NSL),
)
def _edge_agg(h_hbm, src_hbm, dst_hbm, out_hbm, sring, dring, rows_v,
              acc_sh, *sems):
    is_sem = sems[:NSL]
    id_sem = sems[NSL:2 * NSL]
    gsem = sems[2 * NSL:3 * NSL]
    ssem = sems[3 * NSL:]
    cid = lax.axis_index("c")
    sid = lax.axis_index("s")
    wid = sid * NC + cid
    base = wid * EPT

    def issue_src(c, j):
        pltpu.async_copy(src_hbm.at[pl.ds(base + c * CH, CH)], sring.at[j],
                         is_sem[j])

    def issue_dst(c, j):
        pltpu.async_copy(dst_hbm.at[pl.ds(base + c * CH, CH)], dring.at[j],
                         id_sem[j])

    def wait_src(j):
        pltpu.make_async_copy(src_hbm.at[pl.ds(0, CH)], sring.at[j],
                              is_sem[j]).wait()

    def wait_dst(j):
        pltpu.make_async_copy(dst_hbm.at[pl.ds(0, CH)], dring.at[j],
                              id_sem[j]).wait()

    def issue_gather(j):
        pltpu.async_copy(h_hbm.at[sring.at[j]], rows_v.at[j], gsem[j])

    def wait_gather(j):
        pltpu.make_async_copy(h_hbm.at[sring.at[0]], rows_v.at[j],
                              gsem[j]).wait()

    def issue_scatter(j):
        pltpu.async_copy(rows_v.at[j], acc_sh.at[dring.at[j]], ssem[j],
                         add=True)

    def wait_scatter(j):
        pltpu.make_async_copy(rows_v.at[j], acc_sh.at[dring.at[j]],
                              ssem[j]).wait()

    for j in range(NSL):
        issue_src(j, j)
    for j in (0, 1):
        issue_dst(j, j)

    zeros16 = jnp.zeros((16,), jnp.float32)

    @pl.loop(0, CH)
    def _zero_rows(rr):
        @pl.loop(0, D // 16)
        def _zero_cols(cc):
            rows_v[0, rr, pl.ds(cc * 16, 16)] = zeros16

    @pl.loop(0, ZCH)
    def _zero_acc(z):
        pltpu.sync_copy(rows_v.at[0], acc_sh.at[pl.ds(sid * ZPT + z * CH, CH)])

    plsc.subcore_barrier()

    for j in (0, 1):
        wait_src(j)
        issue_gather(j)

    for j in (0, 1):
        wait_gather(j)
        issue_src(4 + j, j)
        wait_dst(j)
        issue_scatter(j)
    for j in (2, 3):
        issue_dst(j, j)
        wait_src(j)
        issue_gather(j)
    for j in (2, 3):
        wait_gather(j)
        issue_src(4 + j, j)
        wait_dst(j)
        issue_scatter(j)
    for j in (0, 1):
        wait_scatter(j)
        issue_dst(4 + j, j)
        wait_src(j)
        issue_gather(j)

    @pl.loop(1, NPAIR - 1)
    def _pairs(p):
        c0 = p * NSL
        for j in (0, 1):
            wait_gather(j)
            issue_src(c0 + 4 + j, j)
            wait_dst(j)
            issue_scatter(j)
        for j in (2, 3):
            wait_scatter(j)
            issue_dst(c0 + j, j)
            wait_src(j)
            issue_gather(j)
        for j in (2, 3):
            wait_gather(j)
            issue_src(c0 + 4 + j, j)
            wait_dst(j)
            issue_scatter(j)
        for j in (0, 1):
            wait_scatter(j)
            issue_dst(c0 + 4 + j, j)
            wait_src(j)
            issue_gather(j)

    c0 = NCH - NSL
    for j in (0, 1):
        wait_gather(j)
        wait_dst(j)
        issue_scatter(j)
    for j in (2, 3):
        wait_scatter(j)
        issue_dst(c0 + j, j)
        wait_src(j)
        issue_gather(j)
    for j in (2, 3):
        wait_gather(j)
        wait_dst(j)
        issue_scatter(j)
    for j in range(NSL):
        wait_scatter(j)

    plsc.subcore_barrier()

    pltpu.sync_copy(acc_sh.at[pl.ds(sid * ZPT, ZPT)],
                    out_hbm.at[cid, pl.ds(sid * ZPT, ZPT)])


BN = 2000
NB = N // BN


def _layer_body(h_ref, p0_ref, p1_ref, w_ref, b_ref, o_ref):
    s = h_ref[...] + p0_ref[...] + p1_ref[...]
    y = jnp.dot(s, w_ref[...], preferred_element_type=jnp.float32) + b_ref[...]
    o_ref[...] = jnp.maximum(y, 0.0)


def _layer_tc(h, p0, p1, W, b2d):
    return pl.pallas_call(
        _layer_body,
        grid=(NB,),
        in_specs=[
            pl.BlockSpec((BN, D), lambda i: (i, 0)),
            pl.BlockSpec((BN, D), lambda i: (i, 0)),
            pl.BlockSpec((BN, D), lambda i: (i, 0)),
            pl.BlockSpec((D, D), lambda i: (0, 0)),
            pl.BlockSpec((1, D), lambda i: (0, 0)),
        ],
        out_specs=pl.BlockSpec((BN, D), lambda i: (i, 0)),
        out_shape=jax.ShapeDtypeStruct((N, D), jnp.float32),
    )(h, p0, p1, W, b2d)


def _final_body(h_ref, p0_ref, p1_ref, w2_ref, b2_ref, batch_ref, wg_ref,
                bg_ref, o_ref, sums, counts):
    i = pl.program_id(0)

    @pl.when(i == 0)
    def _():
        sums[...] = jnp.zeros_like(sums)
        counts[...] = jnp.zeros_like(counts)

    s = h_ref[...] + p0_ref[...] + p1_ref[...]
    h2 = jnp.maximum(
        jnp.dot(s, w2_ref[...], preferred_element_type=jnp.float32)
        + b2_ref[...], 0.0)

    bt = batch_ref[...].reshape(1, BN)
    gidx = lax.broadcasted_iota(jnp.int32, (G, BN), 0)
    P = (bt == gidx).astype(jnp.float32)
    sums[...] += jnp.dot(P, h2, preferred_element_type=jnp.float32)
    counts[...] += jnp.broadcast_to(jnp.sum(P, axis=1, keepdims=True), (G, D))

    @pl.when(i == NB - 1)
    def _():
        hg = sums[...] / jnp.maximum(counts[...], 1.0)
        o_ref[...] = (jnp.dot(hg, wg_ref[...], preferred_element_type=jnp.float32)
                      + bg_ref[...])


def _final_tc(h1, p0, p1, W2, b2d, batch3d, Wg, bg2d):
    return pl.pallas_call(
        _final_body,
        grid=(NB,),
        in_specs=[
            pl.BlockSpec((BN, D), lambda i: (i, 0)),
            pl.BlockSpec((BN, D), lambda i: (i, 0)),
            pl.BlockSpec((BN, D), lambda i: (i, 0)),
            pl.BlockSpec((D, D), lambda i: (0, 0)),
            pl.BlockSpec((1, D), lambda i: (0, 0)),
            pl.BlockSpec((1, 1, BN), lambda i: (i, 0, 0)),
            pl.BlockSpec((D, D), lambda i: (0, 0)),
            pl.BlockSpec((1, D), lambda i: (0, 0)),
        ],
        out_specs=pl.BlockSpec((G, D), lambda i: (0, 0)),
        out_shape=jax.ShapeDtypeStruct((G, D), jnp.float32),
        scratch_shapes=[
            pltpu.VMEM((G, D), jnp.float32),
            pltpu.VMEM((G, D), jnp.float32),
        ],
    )(h1, p0, p1, W2, b2d, batch3d, Wg, bg2d)


def kernel(x, edge_index, batch, W1, b1, W2, b2, Wg, bg):
    pad = EPAD - E
    src = jnp.concatenate(
        [edge_index[0].astype(jnp.int32), jnp.zeros((pad,), jnp.int32)])
    dst = jnp.concatenate(
        [edge_index[1].astype(jnp.int32),
         N + (jnp.arange(pad, dtype=jnp.int32) % (NPAD - N))])
    batch3d = batch.astype(jnp.int32).reshape(NB, 1, BN)

    p = _edge_agg(x, src, dst)
    h1 = _layer_tc(x, p[0, :N], p[1, :N], W1, b1.reshape(1, D))
    q = _edge_agg(h1, src, dst)
    return _final_tc(h1, q[0, :N], q[1, :N], W2, b2.reshape(1, D),
                     batch3d, Wg, bg.reshape(1, D))


# speedup vs baseline: 1.1607x; 1.1606x over previous
import functools

import jax
import jax.numpy as jnp
from jax import lax
from jax.experimental import pallas as pl
from jax.experimental.pallas import tpu as pltpu
from jax.experimental.pallas import tpu_sc as plsc

N = 10000
E = 320000
D = 128
G = 128

NC = 2
NS = 16
NW = NC * NS
CH = 64
NCH = 162
EPT = NCH * CH
EPAD = NW * EPT
NBUF = 3
NGRP = NCH // NBUF
NPAD = 10240
ZPT = NPAD // NS
ZCH = ZPT // CH

_sc_mesh = plsc.VectorSubcoreMesh(
    core_axis_name="c", subcore_axis_name="s", num_cores=NC, num_subcores=NS)


@functools.partial(
    pl.kernel,
    out_type=jax.ShapeDtypeStruct((NC, NPAD, D), jnp.float32),
    mesh=_sc_mesh,
    scratch_types=[
        pltpu.VMEM((NBUF, CH), jnp.int32),
        pltpu.VMEM((NBUF, CH), jnp.int32),
        pltpu.VMEM((NBUF, CH, D), jnp.float32),
        pltpu.VMEM_SHARED((NPAD, D), jnp.float32),
    ] + [pltpu.SemaphoreType.DMA] * (4 * NBUF),
)
def _edge_agg(h_hbm, src_hbm, dst_hbm, out_hbm, sring, dring, rows_v,
              acc_sh, *sems):
    is_sem = sems[:NBUF]
    id_sem = sems[NBUF:2 * NBUF]
    gsem = sems[2 * NBUF:3 * NBUF]
    ssem = sems[3 * NBUF:]
    cid = lax.axis_index("c")
    sid = lax.axis_index("s")
    wid = sid * NC + cid
    base = wid * EPT

    def issue_idx(c, b):
        off = base + c * CH
        pltpu.async_copy(src_hbm.at[pl.ds(off, CH)], sring.at[b], is_sem[b])
        pltpu.async_copy(dst_hbm.at[pl.ds(off, CH)], dring.at[b], id_sem[b])

    def wait_idx_issue_gather(b):
        pltpu.make_async_copy(src_hbm.at[pl.ds(0, CH)], sring.at[b],
                              is_sem[b]).wait()
        pltpu.async_copy(h_hbm.at[sring.at[b]], rows_v.at[b], gsem[b])

    def wait_gather_issue_scatter(b):
        pltpu.make_async_copy(h_hbm.at[sring.at[b]], rows_v.at[b],
                              gsem[b]).wait()
        pltpu.make_async_copy(dst_hbm.at[pl.ds(0, CH)], dring.at[b],
                              id_sem[b]).wait()
        pltpu.async_copy(rows_v.at[b], acc_sh.at[dring.at[b]], ssem[b],
                         add=True)

    def wait_scatter(b):
        pltpu.make_async_copy(rows_v.at[b], acc_sh.at[dring.at[b]],
                              ssem[b]).wait()

    for b in range(NBUF):
        issue_idx(b, b)

    zeros16 = jnp.zeros((16,), jnp.float32)

    @pl.loop(0, CH)
    def _zero_rows(r):
        @pl.loop(0, D // 16)
        def _zero_cols(c):
            rows_v[0, r, pl.ds(c * 16, 16)] = zeros16

    @pl.loop(0, ZCH)
    def _zero_acc(z):
        pltpu.sync_copy(rows_v.at[0], acc_sh.at[pl.ds(sid * ZPT + z * CH, CH)])

    plsc.subcore_barrier()

    @pl.loop(0, NGRP)
    def _groups(g):
        c0 = g * NBUF
        for b in range(NBUF):
            wait_idx_issue_gather(b)
        for b in range(NBUF):
            wait_gather_issue_scatter(b)
        for b in range(NBUF):
            nxt = c0 + NBUF + b

            @pl.when(nxt < NCH)
            def _():
                wait_scatter(b)
                issue_idx(nxt, b)

    leftover = range(NGRP * NBUF, NCH)
    for t in leftover:
        wait_idx_issue_gather(t % NBUF)
    for t in leftover:
        wait_gather_issue_scatter(t % NBUF)
    for b in range(NBUF):
        wait_scatter(b)

    plsc.subcore_barrier()

    pltpu.sync_copy(acc_sh.at[pl.ds(sid * ZPT, ZPT)],
                    out_hbm.at[cid, pl.ds(sid * ZPT, ZPT)])


BN = 2000
NB = N // BN


def _layer_body(h_ref, p0_ref, p1_ref, w_ref, b_ref, o_ref):
    s = h_ref[...] + p0_ref[...] + p1_ref[...]
    y = jnp.dot(s, w_ref[...], preferred_element_type=jnp.float32) + b_ref[...]
    o_ref[...] = jnp.maximum(y, 0.0)


def _layer_tc(h, p0, p1, W, b2d):
    return pl.pallas_call(
        _layer_body,
        grid=(NB,),
        in_specs=[
            pl.BlockSpec((BN, D), lambda i: (i, 0)),
            pl.BlockSpec((BN, D), lambda i: (i, 0)),
            pl.BlockSpec((BN, D), lambda i: (i, 0)),
            pl.BlockSpec((D, D), lambda i: (0, 0)),
            pl.BlockSpec((1, D), lambda i: (0, 0)),
        ],
        out_specs=pl.BlockSpec((BN, D), lambda i: (i, 0)),
        out_shape=jax.ShapeDtypeStruct((N, D), jnp.float32),
    )(h, p0, p1, W, b2d)


def _final_body(h_ref, p0_ref, p1_ref, w2_ref, b2_ref, batch_ref, wg_ref,
                bg_ref, o_ref, sums, counts):
    i = pl.program_id(0)

    @pl.when(i == 0)
    def _():
        sums[...] = jnp.zeros_like(sums)
        counts[...] = jnp.zeros_like(counts)

    s = h_ref[...] + p0_ref[...] + p1_ref[...]
    h2 = jnp.maximum(
        jnp.dot(s, w2_ref[...], preferred_element_type=jnp.float32)
        + b2_ref[...], 0.0)

    bt = batch_ref[...].reshape(1, BN)
    gidx = lax.broadcasted_iota(jnp.int32, (G, BN), 0)
    P = (bt == gidx).astype(jnp.float32)
    sums[...] += jnp.dot(P, h2, preferred_element_type=jnp.float32)
    counts[...] += jnp.broadcast_to(jnp.sum(P, axis=1, keepdims=True), (G, D))

    @pl.when(i == NB - 1)
    def _():
        hg = sums[...] / jnp.maximum(counts[...], 1.0)
        o_ref[...] = (jnp.dot(hg, wg_ref[...], preferred_element_type=jnp.float32)
                      + bg_ref[...])


def _final_tc(h1, p0, p1, W2, b2d, batch3d, Wg, bg2d):
    return pl.pallas_call(
        _final_body,
        grid=(NB,),
        in_specs=[
            pl.BlockSpec((BN, D), lambda i: (i, 0)),
            pl.BlockSpec((BN, D), lambda i: (i, 0)),
            pl.BlockSpec((BN, D), lambda i: (i, 0)),
            pl.BlockSpec((D, D), lambda i: (0, 0)),
            pl.BlockSpec((1, D), lambda i: (0, 0)),
            pl.BlockSpec((1, 1, BN), lambda i: (i, 0, 0)),
            pl.BlockSpec((D, D), lambda i: (0, 0)),
            pl.BlockSpec((1, D), lambda i: (0, 0)),
        ],
        out_specs=pl.BlockSpec((G, D), lambda i: (0, 0)),
        out_shape=jax.ShapeDtypeStruct((G, D), jnp.float32),
        scratch_shapes=[
            pltpu.VMEM((G, D), jnp.float32),
            pltpu.VMEM((G, D), jnp.float32),
        ],
    )(h1, p0, p1, W2, b2d, batch3d, Wg, bg2d)


def kernel(x, edge_index, batch, W1, b1, W2, b2, Wg, bg):
    pad = EPAD - E
    src = jnp.concatenate(
        [edge_index[0].astype(jnp.int32), jnp.zeros((pad,), jnp.int32)])
    dst = jnp.concatenate(
        [edge_index[1].astype(jnp.int32),
         N + (jnp.arange(pad, dtype=jnp.int32) % (NPAD - N))])
    batch3d = batch.astype(jnp.int32).reshape(NB, 1, BN)

    p = _edge_agg(x, src, dst)
    h1 = _layer_tc(x, p[0, :N], p[1, :N], W1, b1.reshape(1, D))
    q = _edge_agg(h1, src, dst)
    return _final_tc(h1, q[0, :N], q[1, :N], W2, b2.reshape(1, D),
                     batch3d, Wg, bg.reshape(1, D))


# speedup vs baseline: 5.1772x; 4.4606x over previous
import functools

import jax
import jax.numpy as jnp
from jax import lax
from jax.experimental import pallas as pl
from jax.experimental.pallas import tpu as pltpu
from jax.experimental.pallas import tpu_sc as plsc

N = 10000
E = 320000
D = 128
G = 128

NC = 2
NS = 16
NW = NC * NS
CH = 64
NCHB = 156
XTRA = (E - NW * NCHB * CH) // CH
NSL = 4
NPAIR = NCHB // NSL
NPAD = 10240
ZPT = NPAD // NS
ZCH = ZPT // CH

_sc_mesh = plsc.VectorSubcoreMesh(
    core_axis_name="c", subcore_axis_name="s", num_cores=NC, num_subcores=NS)


@functools.partial(
    pl.kernel,
    out_type=jax.ShapeDtypeStruct((NC, NPAD, D), jnp.float32),
    mesh=_sc_mesh,
    scratch_types=[
        pltpu.VMEM((NSL, CH), jnp.int32),
        pltpu.VMEM((NSL, CH), jnp.int32),
        pltpu.VMEM((NSL, CH, D), jnp.float32),
        pltpu.VMEM_SHARED((NPAD, D), jnp.float32),
    ] + [pltpu.SemaphoreType.DMA] * (4 * NSL),
)
def _edge_agg(h_hbm, src_hbm, dst_hbm, out_hbm, sring, dring, rows_v,
              acc_sh, *sems):
    is_sem = sems[:NSL]
    id_sem = sems[NSL:2 * NSL]
    gsem = sems[2 * NSL:3 * NSL]
    ssem = sems[3 * NSL:]
    cid = lax.axis_index("c")
    sid = lax.axis_index("s")
    wid = sid * NC + cid
    base = wid * (NCHB * CH) + jnp.minimum(wid, XTRA) * CH

    def issue_src(c, j):
        pltpu.async_copy(src_hbm.at[pl.ds(base + c * CH, CH)], sring.at[j],
                         is_sem[j])

    def issue_dst(c, j):
        pltpu.async_copy(dst_hbm.at[pl.ds(base + c * CH, CH)], dring.at[j],
                         id_sem[j])

    def wait_src(j):
        pltpu.make_async_copy(src_hbm.at[pl.ds(0, CH)], sring.at[j],
                              is_sem[j]).wait()

    def wait_dst(j):
        pltpu.make_async_copy(dst_hbm.at[pl.ds(0, CH)], dring.at[j],
                              id_sem[j]).wait()

    def issue_gather(j):
        pltpu.async_copy(h_hbm.at[sring.at[j]], rows_v.at[j], gsem[j])

    def wait_gather(j):
        pltpu.make_async_copy(h_hbm.at[sring.at[0]], rows_v.at[j],
                              gsem[j]).wait()

    def issue_scatter(j):
        pltpu.async_copy(rows_v.at[j], acc_sh.at[dring.at[j]], ssem[j],
                         add=True)

    def wait_scatter(j):
        pltpu.make_async_copy(rows_v.at[j], acc_sh.at[dring.at[j]],
                              ssem[j]).wait()

    for j in range(NSL):
        issue_src(j, j)
    for j in (0, 1):
        issue_dst(j, j)

    zeros16 = jnp.zeros((16,), jnp.float32)

    @pl.loop(0, CH)
    def _zero_rows(rr):
        @pl.loop(0, D // 16)
        def _zero_cols(cc):
            rows_v[0, rr, pl.ds(cc * 16, 16)] = zeros16

    @pl.loop(0, ZCH)
    def _zero_acc(z):
        pltpu.sync_copy(rows_v.at[0], acc_sh.at[pl.ds(sid * ZPT + z * CH, CH)])

    plsc.subcore_barrier()

    for j in (0, 1):
        wait_src(j)
        issue_gather(j)

    for j in (0, 1):
        wait_gather(j)
        issue_src(4 + j, j)
        wait_dst(j)
        issue_scatter(j)
    for j in (2, 3):
        issue_dst(j, j)
        wait_src(j)
        issue_gather(j)
    for j in (2, 3):
        wait_gather(j)
        issue_src(4 + j, j)
        wait_dst(j)
        issue_scatter(j)
    for j in (0, 1):
        wait_scatter(j)
        issue_dst(4 + j, j)
        wait_src(j)
        issue_gather(j)

    @pl.loop(1, NPAIR - 1)
    def _pairs(p):
        c0 = p * NSL
        for j in (0, 1):
            wait_gather(j)
            issue_src(c0 + 4 + j, j)
            wait_dst(j)
            issue_scatter(j)
        for j in (2, 3):
            wait_scatter(j)
            issue_dst(c0 + j, j)
            wait_src(j)
            issue_gather(j)
        for j in (2, 3):
            wait_gather(j)
            issue_src(c0 + 4 + j, j)
            wait_dst(j)
            issue_scatter(j)
        for j in (0, 1):
            wait_scatter(j)
            issue_dst(c0 + 4 + j, j)
            wait_src(j)
            issue_gather(j)

    c0 = NCHB - NSL
    for j in (0, 1):
        wait_gather(j)
        wait_dst(j)
        issue_scatter(j)
    for j in (2, 3):
        wait_scatter(j)
        issue_dst(c0 + j, j)
        wait_src(j)
        issue_gather(j)
    for j in (2, 3):
        wait_gather(j)
        wait_dst(j)
        issue_scatter(j)
    for j in range(NSL):
        wait_scatter(j)

    @pl.when(wid < XTRA)
    def _extra():
        issue_src(NCHB, 0)
        issue_dst(NCHB, 0)
        wait_src(0)
        issue_gather(0)
        wait_gather(0)
        wait_dst(0)
        issue_scatter(0)
        wait_scatter(0)

    plsc.subcore_barrier()

    pltpu.sync_copy(acc_sh.at[pl.ds(sid * ZPT, ZPT)],
                    out_hbm.at[cid, pl.ds(sid * ZPT, ZPT)])


BN = 2000
NB = N // BN


def _layer_body(h_ref, p0_ref, p1_ref, w_ref, b_ref, o_ref):
    s = h_ref[...] + p0_ref[...] + p1_ref[...]
    y = jnp.dot(s, w_ref[...], preferred_element_type=jnp.float32) + b_ref[...]
    o_ref[...] = jnp.maximum(y, 0.0)


def _layer_tc(h, p0, p1, W, b2d):
    return pl.pallas_call(
        _layer_body,
        grid=(NB,),
        in_specs=[
            pl.BlockSpec((BN, D), lambda i: (i, 0)),
            pl.BlockSpec((BN, D), lambda i: (i, 0)),
            pl.BlockSpec((BN, D), lambda i: (i, 0)),
            pl.BlockSpec((D, D), lambda i: (0, 0)),
            pl.BlockSpec((1, D), lambda i: (0, 0)),
        ],
        out_specs=pl.BlockSpec((BN, D), lambda i: (i, 0)),
        out_shape=jax.ShapeDtypeStruct((N, D), jnp.float32),
    )(h, p0, p1, W, b2d)


def _final_body(h_ref, p0_ref, p1_ref, w2_ref, b2_ref, batch_ref, wg_ref,
                bg_ref, o_ref, sums, counts):
    i = pl.program_id(0)

    @pl.when(i == 0)
    def _():
        sums[...] = jnp.zeros_like(sums)
        counts[...] = jnp.zeros_like(counts)

    s = h_ref[...] + p0_ref[...] + p1_ref[...]
    h2 = jnp.maximum(
        jnp.dot(s, w2_ref[...], preferred_element_type=jnp.float32)
        + b2_ref[...], 0.0)

    bt = batch_ref[...].reshape(1, BN)
    gidx = lax.broadcasted_iota(jnp.int32, (G, BN), 0)
    P = (bt == gidx).astype(jnp.float32)
    sums[...] += jnp.dot(P, h2, preferred_element_type=jnp.float32)
    counts[...] += jnp.broadcast_to(jnp.sum(P, axis=1, keepdims=True), (G, D))

    @pl.when(i == NB - 1)
    def _():
        hg = sums[...] / jnp.maximum(counts[...], 1.0)
        o_ref[...] = (jnp.dot(hg, wg_ref[...], preferred_element_type=jnp.float32)
                      + bg_ref[...])


def _final_tc(h1, p0, p1, W2, b2d, batch3d, Wg, bg2d):
    return pl.pallas_call(
        _final_body,
        grid=(NB,),
        in_specs=[
            pl.BlockSpec((BN, D), lambda i: (i, 0)),
            pl.BlockSpec((BN, D), lambda i: (i, 0)),
            pl.BlockSpec((BN, D), lambda i: (i, 0)),
            pl.BlockSpec((D, D), lambda i: (0, 0)),
            pl.BlockSpec((1, D), lambda i: (0, 0)),
            pl.BlockSpec((1, 1, BN), lambda i: (i, 0, 0)),
            pl.BlockSpec((D, D), lambda i: (0, 0)),
            pl.BlockSpec((1, D), lambda i: (0, 0)),
        ],
        out_specs=pl.BlockSpec((G, D), lambda i: (0, 0)),
        out_shape=jax.ShapeDtypeStruct((G, D), jnp.float32),
        scratch_shapes=[
            pltpu.VMEM((G, D), jnp.float32),
            pltpu.VMEM((G, D), jnp.float32),
        ],
    )(h1, p0, p1, W2, b2d, batch3d, Wg, bg2d)


def kernel(x, edge_index, batch, W1, b1, W2, b2, Wg, bg):
    src = edge_index[0].astype(jnp.int32)
    dst = edge_index[1].astype(jnp.int32)
    batch3d = batch.astype(jnp.int32).reshape(NB, 1, BN)

    p = _edge_agg(x, src, dst)
    h1 = _layer_tc(x, p[0, :N], p[1, :N], W1, b1.reshape(1, D))
    q = _edge_agg(h1, src, dst)
    return _final_tc(h1, q[0, :N], q[1, :N], W2, b2.reshape(1, D),
                     batch3d, Wg, bg.reshape(1, D))
